# baseline (speedup 1.0000x reference)
import jax
import jax.numpy as jnp
from jax.experimental import pallas as pl
from jax.experimental.pallas import tpu as pltpu

_BM1 = 2048
_BM2 = 2048


def _pool_kernel(x_ref, w1, b1, w2, b2, w3, b3, w4, b4, out_ref):
    f32 = jnp.float32
    h = jnp.maximum(jnp.dot(x_ref[...], w1[...], preferred_element_type=f32) + b1[...], 0.0)
    h = jnp.maximum(jnp.dot(h, w2[...], preferred_element_type=f32) + b2[...], 0.0)
    y = jnp.maximum(jnp.dot(h, w3[...], preferred_element_type=f32) + b3[...], 0.0)
    y = jnp.maximum(jnp.dot(y, w4[...], preferred_element_type=f32) + b4[...], 0.0)
    out_ref[0] = jnp.max(y, axis=0, keepdims=True)


def _head_kernel(x_ref, part_ref, w1, b1, w2, b2, w5a, w5b, b5, w6, b6,
                 w7, b7, w8, b8, out_ref):
    f32 = jnp.float32
    g = jnp.max(part_ref[...], axis=0, keepdims=True)
    c5 = jnp.dot(g, w5b[...], preferred_element_type=f32) + b5[...]
    h = jnp.maximum(jnp.dot(x_ref[...], w1[...], preferred_element_type=f32) + b1[...], 0.0)
    h = jnp.maximum(jnp.dot(h, w2[...], preferred_element_type=f32) + b2[...], 0.0)
    z = jnp.maximum(jnp.dot(h, w5a[...], preferred_element_type=f32) + c5, 0.0)
    z = jnp.maximum(jnp.dot(z, w6[...], preferred_element_type=f32) + b6[...], 0.0)
    z = jnp.maximum(jnp.dot(z, w7[...], preferred_element_type=f32) + b7[...], 0.0)
    out_ref[...] = jnp.dot(z, w8[...], preferred_element_type=f32) + b8[...]


def _full(shape):
    return pl.BlockSpec(shape, lambda i: tuple(0 for _ in shape))


def kernel(x, W1, b1, W2, b2, W3, b3, W4, b4, W5, b5, W6, b6, W7, b7, W8, b8):
    n = x.shape[2]
    pts = x.reshape(n, 3)
    w1t, w2t, w3t, w4t = W1.T, W2.T, W3.T, W4.T
    w5t = W5.T
    w5a, w5b = w5t[:64], w5t[64:]
    w6t, w7t, w8t = W6.T, W7.T, W8.T
    b1r, b2r, b3r, b4r = (b.reshape(1, -1) for b in (b1, b2, b3, b4))
    b5r, b6r, b7r, b8r = (b.reshape(1, -1) for b in (b5, b6, b7, b8))

    g1 = n // _BM1
    partials = pl.pallas_call(
        _pool_kernel,
        grid=(g1,),
        in_specs=[
            pl.BlockSpec((_BM1, 3), lambda i: (i, 0)),
            _full((3, 64)), _full((1, 64)),
            _full((64, 64)), _full((1, 64)),
            _full((64, 128)), _full((1, 128)),
            _full((128, 1024)), _full((1, 1024)),
        ],
        out_specs=pl.BlockSpec((1, 1, 1024), lambda i: (i, 0, 0)),
        out_shape=jax.ShapeDtypeStruct((g1, 1, 1024), jnp.float32),
        compiler_params=pltpu.CompilerParams(
            dimension_semantics=("parallel",),
        ),
        name="pointnet_pool",
    )(pts, w1t, b1r, w2t, b2r, w3t, b3r, w4t, b4r)

    part2 = partials.reshape(g1, 1024)

    g2 = n // _BM2
    out = pl.pallas_call(
        _head_kernel,
        grid=(g2,),
        in_specs=[
            pl.BlockSpec((_BM2, 3), lambda i: (i, 0)),
            _full((g1, 1024)),
            _full((3, 64)), _full((1, 64)),
            _full((64, 64)), _full((1, 64)),
            _full((64, 512)), _full((1024, 512)), _full((1, 512)),
            _full((512, 256)), _full((1, 256)),
            _full((256, 128)), _full((1, 128)),
            _full((128, 1)), _full((1, 1)),
        ],
        out_specs=pl.BlockSpec((_BM2, 1), lambda i: (i, 0)),
        out_shape=jax.ShapeDtypeStruct((n, 1), jnp.float32),
        compiler_params=pltpu.CompilerParams(
            dimension_semantics=("parallel",),
        ),
        name="pointnet_head",
    )(pts, part2, w1t, b1r, w2t, b2r, w5a, w5b, b5r, w6t, b6r, w7t, b7r,
      w8t, b8r)

    return out.reshape(1, 1, n, 1)
